# Initial kernel scaffold; baseline (speedup 1.0000x reference)
import jax
import jax.numpy as jnp
from jax.experimental import pallas as pl


def kernel(pos, macro_mask, macro_size_x, macro_size_y):
    raise NotImplementedError("write your pallas kernel here")


# speedup vs baseline: 1.0056x; 1.0056x over previous
import jax
import jax.numpy as jnp
from jax.experimental import pallas as pl
from jax.experimental.pallas import tpu as pltpu

_N = 2048
_NUM_PHYS = 600000
_THRESH = 1.0
_BLK = 256


def _notch_kernel(xc_ref, xr_ref, yc_ref, yr_ref, sxc_ref, sxr_ref,
                  syc_ref, syr_ref, mc_ref, mr_ref, out_ref):
    xr = xr_ref[...]
    yr = yr_ref[...]
    sxr = sxr_ref[...] * 0.5
    syr = syr_ref[...] * 0.5
    mr = mr_ref[...]

    cols = jax.lax.broadcasted_iota(jnp.int32, (_BLK, _N), 1)

    def body(r, acc):
        base = r * _BLK
        xc = xc_ref[pl.ds(base, _BLK), :]
        yc = yc_ref[pl.ds(base, _BLK), :]
        sxc = sxc_ref[pl.ds(base, _BLK), :] * 0.5
        syc = syc_ref[pl.ds(base, _BLK), :] * 0.5
        mc = mc_ref[pl.ds(base, _BLK), :]
        dx = jnp.abs(xc - xr) - (sxc + sxr)
        dy = jnp.abs(yc - yr) - (syc + syr)
        d = jnp.maximum(dx, 0.0) + jnp.maximum(dy, 0.0)
        p = jnp.maximum(_THRESH - d, 0.0)
        rows = base + jax.lax.broadcasted_iota(jnp.int32, (_BLK, _N), 0)
        w = jnp.where(cols > rows, mc * mr, 0.0)
        return acc + jnp.sum(w * (p * p))

    total = jax.lax.fori_loop(0, _N // _BLK, body, jnp.float32(0.0))
    cnt = jnp.sum(mr)
    out_ref[0, 0] = jnp.where(cnt < 2.0, 0.0, total)


def kernel(pos, macro_mask, macro_size_x, macro_size_y):
    x = jax.lax.slice(pos, (0,), (_N,))
    y = jax.lax.slice(pos, (_NUM_PHYS,), (_NUM_PHYS + _N,))
    m = macro_mask.astype(jnp.float32)
    sx = macro_size_x.astype(jnp.float32)
    sy = macro_size_y.astype(jnp.float32)

    col = lambda v: v.reshape(_N, 1)
    row = lambda v: v.reshape(1, _N)
    args = (col(x), row(x), col(y), row(y), col(sx), row(sx),
            col(sy), row(sy), col(m), row(m))

    out = pl.pallas_call(
        _notch_kernel,
        out_shape=jax.ShapeDtypeStruct((1, 1), jnp.float32),
        out_specs=pl.BlockSpec(memory_space=pltpu.SMEM),
    )(*args)
    return out[0, 0]


# speedup vs baseline: 1.1434x; 1.1370x over previous
import jax
import jax.numpy as jnp
from jax.experimental import pallas as pl
from jax.experimental.pallas import tpu as pltpu

_N = 2048
_NUM_PHYS = 600000
_THRESH = 1.0
_BLK = 256
_NSTRIP = _N // _BLK


def _notch_kernel(xc_ref, yc_ref, hxc_ref, hyc_ref,
                  xr_ref, yr_ref, hxr_ref, hyr_ref, out_ref):
    r = pl.program_id(0)
    base = r * _BLK
    xc = xc_ref[...]
    yc = yc_ref[...]
    hxc = hxc_ref[...]
    hyc = hyc_ref[...]

    def block(cb):
        xr = xr_ref[:, pl.ds(cb, _BLK)]
        yr = yr_ref[:, pl.ds(cb, _BLK)]
        hxr = hxr_ref[:, pl.ds(cb, _BLK)]
        hyr = hyr_ref[:, pl.ds(cb, _BLK)]
        dx = jnp.maximum(jnp.abs(xc - xr) - (hxc + hxr), 0.0)
        dy = jnp.maximum(jnp.abs(yc - yr) - (hyc + hyr), 0.0)
        p = jnp.maximum((_THRESH - dx) - dy, 0.0)
        return p * p

    lrow = jax.lax.broadcasted_iota(jnp.int32, (_BLK, _BLK), 0)
    lcol = jax.lax.broadcasted_iota(jnp.int32, (_BLK, _BLK), 1)
    acc = jnp.where(lcol > lrow, block(base), 0.0)

    def body(c, a):
        return a + block(c * _BLK)

    acc = jax.lax.fori_loop(r + 1, _NSTRIP, body, acc)
    out_ref[0, 0, 0] = jnp.sum(acc)


def kernel(pos, macro_mask, macro_size_x, macro_size_y):
    x = jax.lax.slice(pos, (0,), (_N,))
    y = jax.lax.slice(pos, (_NUM_PHYS,), (_NUM_PHYS + _N,))
    m = macro_mask
    neg = jnp.where(m, jnp.float32(0.0), jnp.float32(-1e7))
    hx = macro_size_x.astype(jnp.float32) * 0.5 + neg
    hy = macro_size_y.astype(jnp.float32) * 0.5 + neg

    col = lambda v: v.reshape(_N, 1)
    row = lambda v: v.reshape(1, _N)

    partial = pl.pallas_call(
        _notch_kernel,
        grid=(_NSTRIP,),
        in_specs=[
            pl.BlockSpec((_BLK, 1), lambda r: (r, 0)),
            pl.BlockSpec((_BLK, 1), lambda r: (r, 0)),
            pl.BlockSpec((_BLK, 1), lambda r: (r, 0)),
            pl.BlockSpec((_BLK, 1), lambda r: (r, 0)),
            pl.BlockSpec((1, _N), lambda r: (0, 0)),
            pl.BlockSpec((1, _N), lambda r: (0, 0)),
            pl.BlockSpec((1, _N), lambda r: (0, 0)),
            pl.BlockSpec((1, _N), lambda r: (0, 0)),
        ],
        out_shape=jax.ShapeDtypeStruct((_NSTRIP, 1, 1), jnp.float32),
        out_specs=pl.BlockSpec((1, 1, 1), lambda r: (r, 0, 0),
                               memory_space=pltpu.SMEM),
        compiler_params=pltpu.CompilerParams(
            dimension_semantics=("parallel",)),
    )(col(x), col(y), col(hx), col(hy), row(x), row(y), row(hx), row(hy))

    total = jnp.sum(partial)
    count = jnp.sum(m.astype(jnp.int32))
    return jnp.where(count < 2, jnp.zeros((), jnp.float32), total)
